# baseline (speedup 1.0000x reference)
import functools
import math

import jax
import jax.numpy as jnp
from jax import lax
from jax.experimental import pallas as pl
from jax.experimental.pallas import tpu as pltpu
from jax.experimental.pallas import tpu_sc as plsc

_W = 128
_CHUNK = 80


def _stage_msg_body(x_ref, pos_ref, wtp_ref, m_ref):
    x = x_ref[...]
    pos = pos_ref[...]
    d = x.shape[1]
    p = pos.shape[1]
    m = None
    for j in range(p):
        zj = jnp.dot(x, wtp_ref[pl.ds(j * d, d), :],
                     preferred_element_type=jnp.float32)
        mj = pos[:, j:j + 1] * zj
        m = mj if m is None else m + mj
    m_ref[...] = m


def _make_sc_segsum(n, g, d, nc, ns):
    nw = nc * ns
    nchunks = n // _CHUNK
    trips = (nchunks + nw - 1) // nw
    rows_per_sub = g // ns

    @functools.partial(
        pl.kernel,
        out_type=jax.ShapeDtypeStruct((nc, g, d), jnp.float32),
        mesh=plsc.VectorSubcoreMesh(core_axis_name="c", subcore_axis_name="s"),
        scratch_types=[
            pltpu.VMEM((_CHUNK,), jnp.int32),
            pltpu.VMEM((_CHUNK, d), jnp.float32),
            pltpu.VMEM_SHARED((g, d), jnp.float32),
        ],
    )
    def sc_segsum(m_hbm, batch_hbm, zero_hbm, out_hbm, idx_v, rows_v, table_sh):
        c = lax.axis_index("c")
        s = lax.axis_index("s")
        w = s * nc + c

        @pl.when(s == 0)
        def _zero():
            pltpu.sync_copy(zero_hbm, table_sh)

        plsc.subcore_barrier()

        def _body(t, carry):
            cidx = t * nw + w

            @pl.when(cidx < nchunks)
            def _chunk():
                base = cidx * _CHUNK
                pltpu.sync_copy(batch_hbm.at[pl.ds(base, _CHUNK)], idx_v)
                pltpu.sync_copy(m_hbm.at[pl.ds(base, _CHUNK), :], rows_v)
                pltpu.sync_copy(rows_v, table_sh.at[idx_v], add=True)

            return carry

        lax.fori_loop(0, trips, _body, 0)
        plsc.subcore_barrier()
        pltpu.sync_copy(
            table_sh.at[pl.ds(s * rows_per_sub, rows_per_sub), :],
            out_hbm.at[c, pl.ds(s * rows_per_sub, rows_per_sub), :])

    return sc_segsum


def _stage_c_body(x_ref, batch_ref, wnsc_ref, xv_ref, wvsc_ref, wn2v_ref,
                  part_ref, xvo_ref, out_ref, y2_ref):
    i = pl.program_id(0)
    d = x_ref.shape[1]

    @pl.when(i == 0)
    def _combine():
        sv = jnp.dot(xv_ref[...], wvsc_ref[...],
                     preferred_element_type=jnp.float32) * (1.0 / math.sqrt(d))
        mv = part_ref[0] + part_ref[1]
        mv = mv * jax.nn.sigmoid(mv)
        xvo = (sv + mv) * (1.0 / math.sqrt(2.0))
        xvo_ref[...] = xvo
        y2_ref[...] = jnp.dot(xvo, wn2v_ref[...],
                              preferred_element_type=jnp.float32) * (1.0 / math.sqrt(d))

    x = x_ref[...]
    s = jnp.dot(x, wnsc_ref[...],
                preferred_element_type=jnp.float32) * (1.0 / math.sqrt(d))
    bb = batch_ref[0]
    b = bb.shape[1]
    g = y2_ref.shape[0]
    lo = batch_ref[0, 0, 0]
    hi = batch_ref[0, 0, b - 1]
    g0 = jnp.minimum((lo // 8) * 8, g - _W)
    fits = (hi - g0) < _W

    def _finish(gath):
        out_ref[...] = (s + gath * jax.nn.sigmoid(gath)) * 0.5

    @pl.when(fits)
    def _narrow():
        onehot_t = (jax.lax.broadcasted_iota(jnp.int32, (_W, b), 0) + g0
                    == bb).astype(jnp.float32)
        _finish(jax.lax.dot_general(
            onehot_t, y2_ref[pl.ds(g0, _W), :], (((0,), (0,)), ((), ())),
            preferred_element_type=jnp.float32))

    @pl.when(jnp.logical_not(fits))
    def _wide():
        gath = None
        for k in range(g // _W):
            onehot_t = (jax.lax.broadcasted_iota(jnp.int32, (_W, b), 0)
                        + k * _W == bb).astype(jnp.float32)
            gk = jax.lax.dot_general(
                onehot_t, y2_ref[k * _W:(k + 1) * _W, :],
                (((0,), (0,)), ((), ())),
                preferred_element_type=jnp.float32)
            gath = gk if gath is None else gath + gk
        _finish(gath)


def kernel(x_virtual, x_node, node_pos_sh, batch, W_vsc, W_nsc, W_tp, W_n2v):
    n, d = x_node.shape
    p = node_pos_sh.shape[1]
    g = x_virtual.shape[0]
    avg_nodes = n / g

    B = 10000
    nb = n // B
    assert nb * B == n

    wtp_stack = (W_tp.transpose(1, 0, 2).reshape(p * d, d)
                 * (1.0 / (math.sqrt(d * p) * math.sqrt(avg_nodes))))
    batch3d = batch.reshape(nb, 1, B)
    zero_gd = jnp.zeros((g, d), jnp.float32)

    m = pl.pallas_call(
        _stage_msg_body,
        grid=(nb,),
        in_specs=[
            pl.BlockSpec((B, d), lambda i: (i, 0)),
            pl.BlockSpec((B, p), lambda i: (i, 0)),
            pl.BlockSpec((p * d, d), lambda i: (0, 0)),
        ],
        out_specs=pl.BlockSpec((B, d), lambda i: (i, 0)),
        out_shape=jax.ShapeDtypeStruct((n, d), jnp.float32),
        compiler_params=pltpu.CompilerParams(
            dimension_semantics=("parallel",)),
    )(x_node, node_pos_sh, wtp_stack)

    partials = _make_sc_segsum(n, g, d, 2, 16)(m, batch, zero_gd)

    xvo, x_node_out = pl.pallas_call(
        _stage_c_body,
        grid=(nb,),
        in_specs=[
            pl.BlockSpec((B, d), lambda i: (i, 0)),
            pl.BlockSpec((1, 1, B), lambda i: (i, 0, 0)),
            pl.BlockSpec((d, d), lambda i: (0, 0)),
            pl.BlockSpec((g, d), lambda i: (0, 0)),
            pl.BlockSpec((d, d), lambda i: (0, 0)),
            pl.BlockSpec((d, d), lambda i: (0, 0)),
            pl.BlockSpec((2, g, d), lambda i: (0, 0, 0)),
        ],
        out_specs=(pl.BlockSpec((g, d), lambda i: (0, 0)),
                   pl.BlockSpec((B, d), lambda i: (i, 0))),
        out_shape=(jax.ShapeDtypeStruct((g, d), jnp.float32),
                   jax.ShapeDtypeStruct((n, d), jnp.float32)),
        scratch_shapes=[pltpu.VMEM((g, d), jnp.float32)],
        compiler_params=pltpu.CompilerParams(
            dimension_semantics=("arbitrary",)),
    )(x_node, batch3d, W_nsc, x_virtual, W_vsc, W_n2v, partials)

    return (xvo, x_node_out)


# speedup vs baseline: 1.5771x; 1.5771x over previous
import math

import jax
import jax.numpy as jnp
from jax.experimental import pallas as pl
from jax.experimental.pallas import tpu as pltpu

_W = 128


def _stage_a_body(x_ref, pos_ref, batch_ref, wtp_ref, seg_ref):
    i = pl.program_id(0)

    @pl.when(i == 0)
    def _init():
        seg_ref[...] = jnp.zeros_like(seg_ref)

    x = x_ref[...]
    pos = pos_ref[...]
    d = x.shape[1]
    p = pos.shape[1]
    n_over_g = pl.num_programs(0) * x.shape[0] / seg_ref.shape[0]
    scale = 1.0 / (math.sqrt(d * p) * math.sqrt(n_over_g))
    m = None
    for j in range(p):
        zj = jnp.dot(x, wtp_ref[pl.ds(j * d, d), :],
                     preferred_element_type=jnp.float32)
        mj = pos[:, j:j + 1] * zj
        m = mj if m is None else m + mj
    m = m * scale
    bb = batch_ref[0]
    b = bb.shape[1]
    g = seg_ref.shape[0]
    lo = batch_ref[0, 0, 0]
    hi = batch_ref[0, 0, b - 1]
    g0 = jnp.minimum((lo // 8) * 8, g - _W)
    fits = (hi - g0) < _W

    @pl.when(fits)
    def _narrow():
        onehot_t = (jax.lax.broadcasted_iota(jnp.int32, (_W, b), 0) + g0
                    == bb).astype(jnp.float32)
        seg_ref[pl.ds(g0, _W), :] += jnp.dot(
            onehot_t, m, preferred_element_type=jnp.float32)

    @pl.when(jnp.logical_not(fits))
    def _wide():
        for k in range(g // _W):
            onehot_t = (jax.lax.broadcasted_iota(jnp.int32, (_W, b), 0)
                        + k * _W == bb).astype(jnp.float32)
            seg_ref[k * _W:(k + 1) * _W, :] += jnp.dot(
                onehot_t, m, preferred_element_type=jnp.float32)


def _stage_c_body(x_ref, batch_ref, wnsc_ref, xv_ref, wvsc_ref, wn2v_ref,
                  seg_ref, xvo_ref, out_ref, y2_ref):
    i = pl.program_id(0)
    d = x_ref.shape[1]

    @pl.when(i == 0)
    def _combine():
        sv = jnp.dot(xv_ref[...], wvsc_ref[...],
                     preferred_element_type=jnp.float32) * (1.0 / math.sqrt(d))
        mv = seg_ref[...]
        mv = mv * jax.nn.sigmoid(mv)
        xvo = (sv + mv) * (1.0 / math.sqrt(2.0))
        xvo_ref[...] = xvo
        y2_ref[...] = jnp.dot(xvo, wn2v_ref[...],
                              preferred_element_type=jnp.float32) * (1.0 / math.sqrt(d))

    x = x_ref[...]
    s = jnp.dot(x, wnsc_ref[...],
                preferred_element_type=jnp.float32) * (1.0 / math.sqrt(d))
    bb = batch_ref[0]
    b = bb.shape[1]
    g = y2_ref.shape[0]
    lo = batch_ref[0, 0, 0]
    hi = batch_ref[0, 0, b - 1]
    g0 = jnp.minimum((lo // 8) * 8, g - _W)
    fits = (hi - g0) < _W

    def _finish(gath):
        out_ref[...] = (s + gath * jax.nn.sigmoid(gath)) * 0.5

    @pl.when(fits)
    def _narrow():
        onehot_t = (jax.lax.broadcasted_iota(jnp.int32, (_W, b), 0) + g0
                    == bb).astype(jnp.float32)
        _finish(jax.lax.dot_general(
            onehot_t, y2_ref[pl.ds(g0, _W), :], (((0,), (0,)), ((), ())),
            preferred_element_type=jnp.float32))

    @pl.when(jnp.logical_not(fits))
    def _wide():
        gath = None
        for k in range(g // _W):
            onehot_t = (jax.lax.broadcasted_iota(jnp.int32, (_W, b), 0)
                        + k * _W == bb).astype(jnp.float32)
            gk = jax.lax.dot_general(
                onehot_t, y2_ref[k * _W:(k + 1) * _W, :],
                (((0,), (0,)), ((), ())),
                preferred_element_type=jnp.float32)
            gath = gk if gath is None else gath + gk
        _finish(gath)


def kernel(x_virtual, x_node, node_pos_sh, batch, W_vsc, W_nsc, W_tp, W_n2v):
    n, d = x_node.shape
    p = node_pos_sh.shape[1]
    g = x_virtual.shape[0]

    B = 10000
    nb = n // B
    assert nb * B == n

    wtp_stack = W_tp.transpose(1, 0, 2).reshape(p * d, d)
    batch3d = batch.reshape(nb, 1, B)

    seg = pl.pallas_call(
        _stage_a_body,
        grid=(nb,),
        in_specs=[
            pl.BlockSpec((B, d), lambda i: (i, 0)),
            pl.BlockSpec((B, p), lambda i: (i, 0)),
            pl.BlockSpec((1, 1, B), lambda i: (i, 0, 0)),
            pl.BlockSpec((p * d, d), lambda i: (0, 0)),
        ],
        out_specs=pl.BlockSpec((g, d), lambda i: (0, 0)),
        out_shape=jax.ShapeDtypeStruct((g, d), jnp.float32),
        compiler_params=pltpu.CompilerParams(
            dimension_semantics=("arbitrary",)),
    )(x_node, node_pos_sh, batch3d, wtp_stack)

    xvo, x_node_out = pl.pallas_call(
        _stage_c_body,
        grid=(nb,),
        in_specs=[
            pl.BlockSpec((B, d), lambda i: (i, 0)),
            pl.BlockSpec((1, 1, B), lambda i: (i, 0, 0)),
            pl.BlockSpec((d, d), lambda i: (0, 0)),
            pl.BlockSpec((g, d), lambda i: (0, 0)),
            pl.BlockSpec((d, d), lambda i: (0, 0)),
            pl.BlockSpec((d, d), lambda i: (0, 0)),
            pl.BlockSpec((g, d), lambda i: (0, 0)),
        ],
        out_specs=(pl.BlockSpec((g, d), lambda i: (0, 0)),
                   pl.BlockSpec((B, d), lambda i: (i, 0))),
        out_shape=(jax.ShapeDtypeStruct((g, d), jnp.float32),
                   jax.ShapeDtypeStruct((n, d), jnp.float32)),
        scratch_shapes=[pltpu.VMEM((g, d), jnp.float32)],
        compiler_params=pltpu.CompilerParams(
            dimension_semantics=("arbitrary",)),
    )(x_node, batch3d, W_nsc, x_virtual, W_vsc, W_n2v, seg)

    return (xvo, x_node_out)
